# Initial kernel scaffold; baseline (speedup 1.0000x reference)
import jax
import jax.numpy as jnp
from jax.experimental import pallas as pl


def kernel(position_ids, table):
    raise NotImplementedError("write your pallas kernel here")


# speedup vs baseline: 1.9796x; 1.9796x over previous
import functools

import jax
import jax.numpy as jnp
from jax import lax
from jax.experimental import pallas as pl
from jax.experimental.pallas import tpu as pltpu
from jax.experimental.pallas import tpu_sc as plsc

_BATCH = 4
_SEQ = 8192
_D = 1024
_ROWS = _BATCH * _SEQ
_C = 32
_TOTAL_CHUNKS = _ROWS // _C


@functools.partial(jax.jit, static_argnums=(2, 3))
def _sc_gather(ids2d, table, nc, ns):
    nw = nc * ns
    ch_w = _TOTAL_CHUNKS // nw

    mesh = plsc.VectorSubcoreMesh(core_axis_name="c", subcore_axis_name="s")

    @functools.partial(
        pl.kernel,
        mesh=mesh,
        out_type=jax.ShapeDtypeStruct((_ROWS, _D), jnp.float32),
        scratch_types=[
            pltpu.VMEM((ch_w, _C), jnp.int32),
            pltpu.VMEM((_C, _D), jnp.float32),
            pltpu.SemaphoreType.DMA,
        ],
    )
    def k(ids_hbm, table_hbm, out_hbm, idx_v, buf, gsem):
        wid = lax.axis_index("s") * nc + lax.axis_index("c")
        base_chunk = wid * ch_w
        pltpu.sync_copy(ids_hbm.at[pl.ds(base_chunk, ch_w)], idx_v)

        def chunk_body(c, carry):
            pltpu.async_copy(table_hbm.at[idx_v.at[c]], buf, gsem).wait()
            pltpu.sync_copy(buf, out_hbm.at[pl.ds((base_chunk + c) * _C, _C)])
            return carry

        lax.fori_loop(0, ch_w, chunk_body, 0)

    return k(ids2d, table)


def kernel(position_ids, table):
    info = plsc.get_sparse_core_info()
    ids2d = position_ids.reshape(_TOTAL_CHUNKS, _C)
    out = _sc_gather(ids2d, table, int(info.num_cores), int(info.num_subcores))
    return out.reshape(_BATCH, _SEQ, _D)


# speedup vs baseline: 2.3766x; 1.2006x over previous
import functools

import jax
import jax.numpy as jnp
from jax import lax
from jax.experimental import pallas as pl
from jax.experimental.pallas import tpu as pltpu
from jax.experimental.pallas import tpu_sc as plsc

_BATCH = 4
_SEQ = 8192
_D = 1024
_ROWS = _BATCH * _SEQ
_C = 32
_TOTAL_CHUNKS = _ROWS // _C
_NBUF = 2


@functools.partial(jax.jit, static_argnums=(2, 3))
def _sc_gather(ids2d, table, nc, ns):
    nw = nc * ns
    ch_w = _TOTAL_CHUNKS // nw
    assert ch_w % _NBUF == 0 and ch_w >= 2 * _NBUF

    mesh = plsc.VectorSubcoreMesh(core_axis_name="c", subcore_axis_name="s")

    @functools.partial(
        pl.kernel,
        mesh=mesh,
        out_type=jax.ShapeDtypeStruct((_ROWS, _D), jnp.float32),
        scratch_types=[
            pltpu.VMEM((ch_w, _C), jnp.int32),
            pltpu.VMEM((_NBUF, _C, _D), jnp.float32),
            pltpu.SemaphoreType.DMA,
            pltpu.SemaphoreType.DMA,
            pltpu.SemaphoreType.DMA,
            pltpu.SemaphoreType.DMA,
        ],
    )
    def k(ids_hbm, table_hbm, out_hbm, idx_v, bufs, g0, g1, s0, s1):
        gsem = (g0, g1)
        ssem = (s0, s1)
        wid = lax.axis_index("s") * nc + lax.axis_index("c")
        base_chunk = wid * ch_w
        pltpu.sync_copy(ids_hbm.at[pl.ds(base_chunk, ch_w)], idx_v)

        def gather(c, b):
            return pltpu.make_async_copy(
                table_hbm.at[idx_v.at[c]], bufs.at[b], gsem[b])

        def scatter(c, b):
            return pltpu.make_async_copy(
                bufs.at[b], out_hbm.at[pl.ds((base_chunk + c) * _C, _C)],
                ssem[b])

        for b in range(_NBUF):
            gather(b, b).start()

        def pair_body(g, carry):
            for b in range(_NBUF):
                c = g * _NBUF + b
                gather(c, b).wait()
                scatter(c, b).start()
                scatter(c, b).wait()
                gather(c + _NBUF, b).start()
            return carry

        lax.fori_loop(0, ch_w // _NBUF - 1, pair_body, 0)

        for b in range(_NBUF):
            c = ch_w - _NBUF + b
            gather(c, b).wait()
            scatter(c, b).start()
            scatter(c, b).wait()

    return k(ids2d, table)


def kernel(position_ids, table):
    info = plsc.get_sparse_core_info()
    ids2d = position_ids.reshape(_TOTAL_CHUNKS, _C)
    out = _sc_gather(ids2d, table, int(info.num_cores), int(info.num_subcores))
    return out.reshape(_BATCH, _SEQ, _D)
